# Initial kernel scaffold; baseline (speedup 1.0000x reference)
import jax
import jax.numpy as jnp
from jax.experimental import pallas as pl


def kernel(src, pos, src_shape, src_start_idx, ref_windows, score_mask, params):
    raise NotImplementedError("write your pallas kernel here")


# speedup vs baseline: 1.0139x; 1.0139x over previous
import math

import jax
import jax.numpy as jnp
import numpy as np
from jax.experimental import pallas as pl
from jax.experimental.pallas import tpu as pltpu

B = 2
N = 35344
D = 256
NH = 8
NL = 1
DFF = 512
KS = 5
NP_ = KS * KS
KEEP = 0.2
HGRID = 188
WGRID = 188
HSIZE = 188.0
DH = D // NH
FG = math.ceil(N * KEEP)
QN = math.ceil(FG * KEEP)


def _mk_kernel_indices():
    start = -(KS - 1) / 2
    end = (KS - 1) / 2
    indices = np.linspace(start, end, KS)
    i, j = np.meshgrid(indices, indices, indexing='ij')
    kern = np.stack([j, i], axis=-1).reshape(-1, 2) / KS
    return jnp.asarray(kern, jnp.float32)


def _ln(x, g, b, eps=1e-5):
    mu = x.mean(-1, keepdims=True)
    var = ((x - mu) ** 2).mean(-1, keepdims=True)
    return (x - mu) / jnp.sqrt(var + eps) * g + b


def _mha_small(q, k, v, p):
    wq, wk, wv = jnp.split(p['in_proj_w'], 3, axis=0)
    bq, bk, bv = jnp.split(p['in_proj_b'], 3, axis=0)
    Bq, L, _ = q.shape

    def proj(x, w, bb):
        return (x @ w.T + bb).reshape(Bq, -1, NH, DH).transpose(0, 2, 1, 3)

    qh = proj(q, wq, bq)
    kh = proj(k, wk, bk)
    vh = proj(v, wv, bv)
    attn = jax.nn.softmax(qh @ kh.transpose(0, 1, 3, 2) / np.sqrt(DH), axis=-1)
    out = (attn @ vh).transpose(0, 2, 1, 3).reshape(Bq, L, D)
    return out @ p['mha_out_w'].T + p['mha_out_b']


def _where_to_attend(query, ref_windows, p, kernel_indices):
    Bq, L, _ = query.shape
    offset_boxes = (query @ p['linear_box_w'].T + p['linear_box_b']).reshape(
        Bq, L, NH, NL, 4)
    rw = ref_windows[:, :, None, None, :]
    ref_boxes = rw[..., jnp.array([0, 1, 3, 4])]
    ref_angles = rw[..., 6:7]
    angles = jnp.broadcast_to(ref_angles, (Bq, L, NH, NL, 1))
    boxes = ref_boxes + offset_boxes / 8.0 * ref_boxes[..., jnp.array([2, 3, 2, 3])]
    boxes = boxes[..., None, :]
    center = boxes[..., :2]
    size = boxes[..., 2:]
    cos_a = jnp.cos(angles)
    sin_a = jnp.sin(angles)
    rot = jnp.stack([cos_a, -sin_a, sin_a, cos_a], axis=-1).reshape(
        Bq, L, NH, NL, 1, 2, 2)
    samp = (query @ p['samp_off_w'].T + p['samp_off_b']).reshape(
        Bq, L, NH, NL, NP_, 2)
    deform = samp / HSIZE
    fixed = kernel_indices * jax.nn.relu(size)
    fixed = center + (fixed[..., None, :] * rot).sum(-1)
    return fixed + deform


def _box_attn(query, value, ref_windows, p):
    Bq, L, _ = query.shape
    S = value.shape[1]
    kernel_indices = _mk_kernel_indices()
    v = (value @ p['value_proj_w'].T + p['value_proj_b']).reshape(Bq, S, NH, DH)
    attn = (query @ p['attn_w_w'].T + p['attn_w_b']).reshape(Bq, L, NH, NL * NP_)
    attn = jax.nn.softmax(attn, axis=-1).reshape(Bq, L, NH, NL, NP_)
    grid = _where_to_attend(query, ref_windows, p, kernel_indices)
    h, w = HGRID, WGRID
    loc = grid[:, :, :, 0]
    ix = loc[..., 0] * w - 0.5
    iy = loc[..., 1] * h - 0.5
    x0 = jnp.floor(ix)
    y0 = jnp.floor(iy)
    x1 = x0 + 1.0
    y1 = y0 + 1.0
    wx1 = ix - x0
    wx0 = 1.0 - wx1
    wy1 = iy - y0
    wy0 = 1.0 - wy1
    v_t = v.transpose(0, 2, 1, 3)

    def gather(xi, yi):
        inb = (xi >= 0) & (xi <= w - 1) & (yi >= 0) & (yi <= h - 1)
        xc = jnp.clip(xi, 0, w - 1).astype(jnp.int32)
        yc = jnp.clip(yi, 0, h - 1).astype(jnp.int32)
        flat = (yc * w + xc).transpose(0, 2, 1, 3).reshape(Bq, NH, L * NP_)
        idx = jnp.broadcast_to(flat[..., None], (Bq, NH, L * NP_, DH))
        g = jnp.take_along_axis(v_t, idx, axis=2).reshape(Bq, NH, L, NP_, DH)
        m = inb.transpose(0, 2, 1, 3).astype(v.dtype)[..., None]
        return g * m

    def wgt(a):
        return a.transpose(0, 2, 1, 3)[..., None]

    sampled = gather(x0, y0) * wgt(wx0 * wy0)
    sampled = sampled + gather(x1, y0) * wgt(wx1 * wy0)
    sampled = sampled + gather(x0, y1) * wgt(wx0 * wy1)
    sampled = sampled + gather(x1, y1) * wgt(wx1 * wy1)
    aw = attn[:, :, :, 0].transpose(0, 2, 1, 3)[..., None]
    out = (sampled * aw).sum(3).transpose(0, 2, 1, 3).reshape(Bq, L, D)
    out = out @ p['out_proj_w'].T + p['out_proj_b']
    return out, attn


def _gat(x, idx):
    ib = jnp.broadcast_to(idx[..., None], idx.shape + (x.shape[-1],))
    return jnp.take_along_axis(x, ib, axis=1)


LPAD = 7168
ROWB = 512


def _ffn_ln_body(x_ref, w1_ref, b1_ref, w2_ref, b2_ref, g_ref, bb_ref, o_ref):
    x = x_ref[...]
    h = jnp.maximum(
        jnp.dot(x, w1_ref[...], preferred_element_type=jnp.float32)
        + b1_ref[...], 0.0)
    y = jnp.dot(h, w2_ref[...], preferred_element_type=jnp.float32) + b2_ref[...]
    s = x + y
    mu = s.mean(-1, keepdims=True)
    var = ((s - mu) ** 2).mean(-1, keepdims=True)
    o_ref[...] = (s - mu) / jnp.sqrt(var + 1e-5) * g_ref[...] + bb_ref[...]


def _ffn_ln(x, p):
    Bq, L, _ = x.shape
    xp = jnp.pad(x, ((0, 0), (0, LPAD - L), (0, 0)))
    xp = xp.reshape(Bq * LPAD // ROWB, ROWB, D)
    out = pl.pallas_call(
        _ffn_ln_body,
        out_shape=jax.ShapeDtypeStruct((Bq * LPAD // ROWB, ROWB, D), jnp.float32),
        grid=(Bq * LPAD // ROWB,),
        in_specs=[
            pl.BlockSpec((1, ROWB, D), lambda i: (i, 0, 0)),
            pl.BlockSpec((D, DFF), lambda i: (0, 0)),
            pl.BlockSpec((DFF,), lambda i: (0,)),
            pl.BlockSpec((DFF, D), lambda i: (0, 0)),
            pl.BlockSpec((D,), lambda i: (0,)),
            pl.BlockSpec((D,), lambda i: (0,)),
            pl.BlockSpec((D,), lambda i: (0,)),
        ],
        out_specs=pl.BlockSpec((1, ROWB, D), lambda i: (i, 0, 0)),
    )(xp, p['lin1_w'].T, p['lin1_b'], p['lin2_w'].T, p['lin2_b'],
      p['norm2_g'], p['norm2_b'])
    return out.reshape(Bq, LPAD, D)[:, :L]


def kernel(src, pos, src_shape, src_start_idx, ref_windows, score_mask, params):
    p = params
    Bq, Ntok, _ = src.shape
    sel_score, indices = jax.lax.top_k(score_mask, FG)
    select_src = _gat(src, indices)
    select_pos = _gat(pos, indices)
    select_ref = _gat(ref_windows, indices)
    query_src = select_src[:, :QN]
    query_pos = select_pos[:, :QN]
    q = query_src + query_pos
    q2 = _mha_small(q, q, query_src, p)
    query_src = _ln(query_src + q2, p['query_norm_g'], p['query_norm_b'])
    select_src = jnp.concatenate([query_src, select_src[:, QN:]], axis=1)
    src2, _ = _box_attn(select_src + select_pos, src, select_ref, p)
    select_src = _ln(select_src + src2, p['norm1_g'], p['norm1_b'])
    select_src = _ffn_ln(select_src, p)
    bidx = jnp.arange(Bq)[:, None]
    output = src.at[bidx, indices].set(select_src)
    return output


# speedup vs baseline: 6.6031x; 6.5128x over previous
import functools
import math

import jax
import jax.numpy as jnp
import numpy as np
from jax import lax
from jax.experimental import pallas as pl
from jax.experimental.pallas import tpu as pltpu
from jax.experimental.pallas import tpu_sc as plsc

B = 2
N = 35344
D = 256
NH = 8
NL = 1
DFF = 512
KS = 5
NP_ = KS * KS
KEEP = 0.2
HGRID = 188
WGRID = 188
HSIZE = 188.0
DH = D // NH
FG = math.ceil(N * KEEP)
QN = math.ceil(FG * KEEP)

LQP = 7168
NPTS = 4 * NP_
TOTQ = B * LQP * NH
NTILES = 32
QPT = TOTQ // NTILES
CQ = 8
NCHUNK = QPT // CQ


def _kern_pts():
    start = -(KS - 1) / 2
    end = (KS - 1) / 2
    idx = np.linspace(start, end, KS)
    i, j = np.meshgrid(idx, idx, indexing='ij')
    kern = np.stack([j, i], axis=-1).reshape(-1, 2) / KS
    return kern.astype(np.float32)


def _kxy(shape_like):
    pi = jax.lax.broadcasted_iota(jnp.int32, (1, NP_), 1)
    kx = ((pi % KS) - (KS - 1) // 2).astype(jnp.float32) / KS
    ky = ((pi // KS) - (KS - 1) // 2).astype(jnp.float32) / KS
    return kx, ky


def _ln(x, g, b, eps=1e-5):
    mu = x.mean(-1, keepdims=True)
    var = ((x - mu) ** 2).mean(-1, keepdims=True)
    return (x - mu) / jnp.sqrt(var + eps) * g + b


def _mha_small(q, k, v, p):
    wq, wk, wv = jnp.split(p['in_proj_w'], 3, axis=0)
    bq, bk, bv = jnp.split(p['in_proj_b'], 3, axis=0)
    Bq, L, _ = q.shape

    def proj(x, w, bb):
        return (x @ w.T + bb).reshape(Bq, -1, NH, DH).transpose(0, 2, 1, 3)

    qh = proj(q, wq, bq)
    kh = proj(k, wk, bk)
    vh = proj(v, wv, bv)
    attn = jax.nn.softmax(qh @ kh.transpose(0, 1, 3, 2) / np.sqrt(DH), axis=-1)
    out = (attn @ vh).transpose(0, 2, 1, 3).reshape(Bq, L, D)
    return out @ p['mha_out_w'].T + p['mha_out_b']


def _gat(x, idx):
    ib = jnp.broadcast_to(idx[..., None], idx.shape + (x.shape[-1],))
    return jnp.take_along_axis(x, ib, axis=1)


RB = 32


def _idxwgt_body(q_ref, ref_ref, wb_ref, bb_ref, wa_ref, ba_ref,
                 wsx_ref, bsx_ref, wsy_ref, bsy_ref, idx_ref, wgt_ref):
    b = pl.program_id(0)
    q = q_ref[0]
    ob = jnp.dot(q, wb_ref[...], preferred_element_type=jnp.float32) + bb_ref[...]
    al = jnp.dot(q, wa_ref[...], preferred_element_type=jnp.float32) + ba_ref[...]
    spx = jnp.dot(q, wsx_ref[...], preferred_element_type=jnp.float32) + bsx_ref[...]
    spy = jnp.dot(q, wsy_ref[...], preferred_element_type=jnp.float32) + bsy_ref[...]
    refw = ref_ref[0]

    rcx, rcy = refw[:, 0:1], refw[:, 1:2]
    rw, rh, ang = refw[:, 3:4], refw[:, 4:5], refw[:, 6:7]
    ca = jnp.cos(ang)
    sa = jnp.sin(ang)
    kx, ky = _kxy(None)
    boff = b * (N * 2)

    idx_parts = []
    wgt_parts = []
    for h in range(NH):
        alh = al[:, h * NP_:(h + 1) * NP_]
        alh = alh - jnp.max(alh, axis=1, keepdims=True)
        ale = jnp.exp(alh)
        aw = ale / jnp.sum(ale, axis=1, keepdims=True)

        obx = ob[:, h * 4 + 0:h * 4 + 1]
        oby = ob[:, h * 4 + 1:h * 4 + 2]
        obw = ob[:, h * 4 + 2:h * 4 + 3]
        obh = ob[:, h * 4 + 3:h * 4 + 4]
        bx = rcx + obx / 8.0 * rw
        by = rcy + oby / 8.0 * rh
        bw = rw + obw / 8.0 * rw
        bh = rh + obh / 8.0 * rh
        sw = jnp.maximum(bw, 0.0)
        sh = jnp.maximum(bh, 0.0)
        fx = kx * sw
        fy = ky * sh
        gx = bx + fx * ca - fy * sa + spx[:, h * NP_:(h + 1) * NP_] / HSIZE
        gy = by + fx * sa + fy * ca + spy[:, h * NP_:(h + 1) * NP_] / HSIZE

        ix = gx * WGRID - 0.5
        iy = gy * HGRID - 0.5
        x0 = jnp.floor(ix)
        y0 = jnp.floor(iy)
        wx1 = ix - x0
        wx0 = 1.0 - wx1
        wy1 = iy - y0
        wy0 = 1.0 - wy1

        def corner(xi, yi, wx, wy):
            inb = ((xi >= 0.0) & (xi <= WGRID - 1.0)
                   & (yi >= 0.0) & (yi <= HGRID - 1.0))
            xc = jnp.clip(xi, 0.0, WGRID - 1.0).astype(jnp.int32)
            yc = jnp.clip(yi, 0.0, HGRID - 1.0).astype(jnp.int32)
            cell = yc * WGRID + xc
            fid = boff + cell * 2 + (h // 4)
            w = aw * wx * wy * inb.astype(jnp.float32)
            return fid, w

        i00, w00 = corner(x0, y0, wx0, wy0)
        i10, w10 = corner(x0 + 1.0, y0, wx1, wy0)
        i01, w01 = corner(x0, y0 + 1.0, wx0, wy1)
        i11, w11 = corner(x0 + 1.0, y0 + 1.0, wx1, wy1)
        idx_parts += [i00, i10, i01, i11]
        wgt_parts += [w00, w10, w01, w11]
    idx_ref[0] = jnp.concatenate(idx_parts, axis=1)
    wgt_ref[0] = jnp.concatenate(wgt_parts, axis=1)


def _idx_wgt(query, refp, p):
    grid = (B, LQP // RB)
    wsamp = p['samp_off_w'].reshape(NH * NP_, 2, D)
    bsamp = p['samp_off_b'].reshape(NH * NP_, 2)
    out = pl.pallas_call(
        _idxwgt_body,
        out_shape=(
            jax.ShapeDtypeStruct((B, LQP, NH * NPTS), jnp.int32),
            jax.ShapeDtypeStruct((B, LQP, NH * NPTS), jnp.float32),
        ),
        grid=grid,
        in_specs=[
            pl.BlockSpec((1, RB, D), lambda b, i: (b, i, 0)),
            pl.BlockSpec((1, RB, 128), lambda b, i: (b, i, 0)),
            pl.BlockSpec((D, NH * 4), lambda b, i: (0, 0)),
            pl.BlockSpec((NH * 4,), lambda b, i: (0,)),
            pl.BlockSpec((D, NH * NP_), lambda b, i: (0, 0)),
            pl.BlockSpec((NH * NP_,), lambda b, i: (0,)),
            pl.BlockSpec((D, NH * NP_), lambda b, i: (0, 0)),
            pl.BlockSpec((NH * NP_,), lambda b, i: (0,)),
            pl.BlockSpec((D, NH * NP_), lambda b, i: (0, 0)),
            pl.BlockSpec((NH * NP_,), lambda b, i: (0,)),
        ],
        out_specs=(
            pl.BlockSpec((1, RB, NH * NPTS), lambda b, i: (b, i, 0)),
            pl.BlockSpec((1, RB, NH * NPTS), lambda b, i: (b, i, 0)),
        ),
    )(query, refp,
      p['linear_box_w'].T, p['linear_box_b'],
      p['attn_w_w'].T, p['attn_w_b'],
      wsamp[:, 0, :].T, bsamp[:, 0],
      wsamp[:, 1, :].T, bsamp[:, 1])
    return out


def _sc_bag(vt, idxs, wgts):
    mesh = plsc.VectorSubcoreMesh(core_axis_name="c", subcore_axis_name="s")

    @functools.partial(
        pl.kernel, mesh=mesh,
        compiler_params=pltpu.CompilerParams(needs_layout_passes=False),
        out_type=jax.ShapeDtypeStruct((TOTQ, DH), jnp.float32),
        scratch_types=[
            pltpu.VMEM((CQ, NPTS), jnp.int32),
            pltpu.VMEM((CQ * NPTS,), jnp.float32),
            pltpu.VMEM((CQ * NPTS, 128), jnp.float32),
            pltpu.VMEM((CQ, DH), jnp.float32),
            pltpu.SemaphoreType.DMA,
        ],
    )
    def bag(vt_hbm, idx_hbm, wgt_hbm, out_hbm, idx_v, wgt_v, rows_v, out_v,
            gsem):
        wid = lax.axis_index("s") * 2 + lax.axis_index("c")
        base = wid * QPT

        def chunk(c, _):
            off = base + c * CQ
            pltpu.sync_copy(idx_hbm.at[pl.ds(off, CQ)], idx_v)
            pltpu.sync_copy(wgt_hbm.at[pl.ds(off * NPTS, CQ * NPTS)], wgt_v)
            copies = []
            for q in range(CQ):
                copies.append(pltpu.async_copy(
                    vt_hbm.at[idx_v.at[q]],
                    rows_v.at[pl.ds(q * NPTS, NPTS)], gsem))
            for cp in copies:
                cp.wait()

            def qbody(q, _):
                rb = q * NPTS
                hoff = (q % 4) * DH
                qv = jnp.full((16,), q * NPTS, jnp.int32)
                acc0 = jnp.zeros((16,), jnp.float32)
                acc1 = jnp.zeros((16,), jnp.float32)
                for j in range(NPTS):
                    wv = plsc.load_gather(wgt_v, [qv + j])
                    acc0 = acc0 + wv * rows_v[rb + j, pl.ds(hoff, 16)]
                    acc1 = acc1 + wv * rows_v[rb + j, pl.ds(hoff + 16, 16)]
                out_v[q, pl.ds(0, 16)] = acc0
                out_v[q, pl.ds(16, 16)] = acc1
                return 0

            lax.fori_loop(0, CQ, qbody, 0)
            pltpu.sync_copy(out_v, out_hbm.at[pl.ds(off, CQ)])
            return 0

        lax.fori_loop(0, NCHUNK, chunk, 0)

    return bag(vt, idxs, wgts)


RB2 = 512


def _tail_body(sel_ref, bag_ref, wo_ref, bo_ref, g1_ref, b1_ref,
               w1_ref, bb1_ref, w2_ref, bb2_ref, g2_ref, b2_ref, o_ref):
    bagp = jnp.dot(bag_ref[0], wo_ref[...],
                   preferred_element_type=jnp.float32) + bo_ref[...]
    x = sel_ref[0] + bagp
    mu = x.mean(-1, keepdims=True)
    var = ((x - mu) ** 2).mean(-1, keepdims=True)
    x = (x - mu) / jnp.sqrt(var + 1e-5) * g1_ref[...] + b1_ref[...]
    h = jnp.maximum(
        jnp.dot(x, w1_ref[...], preferred_element_type=jnp.float32)
        + bb1_ref[...], 0.0)
    y = x + jnp.dot(h, w2_ref[...], preferred_element_type=jnp.float32) \
        + bb2_ref[...]
    mu = y.mean(-1, keepdims=True)
    var = ((y - mu) ** 2).mean(-1, keepdims=True)
    o_ref[0] = (y - mu) / jnp.sqrt(var + 1e-5) * g2_ref[...] + b2_ref[...]


def _tail(sel, bag, p):
    grid = (B * LQP // RB2,)
    sel = sel.reshape(B * LQP // RB2, RB2, D)
    bag = bag.reshape(B * LQP // RB2, RB2, D)
    vec = lambda: pl.BlockSpec((D,), lambda i: (0,))
    mat = lambda s: pl.BlockSpec(s, lambda i: (0, 0))
    out = pl.pallas_call(
        _tail_body,
        out_shape=jax.ShapeDtypeStruct((B * LQP // RB2, RB2, D), jnp.float32),
        grid=grid,
        in_specs=[
            pl.BlockSpec((1, RB2, D), lambda i: (i, 0, 0)),
            pl.BlockSpec((1, RB2, D), lambda i: (i, 0, 0)),
            mat((D, D)), vec(), vec(), vec(),
            mat((D, DFF)), pl.BlockSpec((DFF,), lambda i: (0,)),
            mat((DFF, D)), vec(), vec(), vec(),
        ],
        out_specs=pl.BlockSpec((1, RB2, D), lambda i: (i, 0, 0)),
    )(sel, bag, p['out_proj_w'].T, p['out_proj_b'],
      p['norm1_g'], p['norm1_b'],
      p['lin1_w'].T, p['lin1_b'], p['lin2_w'].T, p['lin2_b'],
      p['norm2_g'], p['norm2_b'])
    return out.reshape(B, LQP, D)


def kernel(src, pos, src_shape, src_start_idx, ref_windows, score_mask,
           params):
    p = params
    Bq = src.shape[0]
    sel_score, indices = jax.lax.top_k(score_mask, FG)
    select_src = _gat(src, indices)
    select_pos = _gat(pos, indices)
    select_ref = _gat(ref_windows, indices)
    query_src = select_src[:, :QN]
    query_pos = select_pos[:, :QN]
    q = query_src + query_pos
    q2 = _mha_small(q, q, query_src, p)
    query_src = _ln(query_src + q2, p['query_norm_g'], p['query_norm_b'])
    select_src = jnp.concatenate([query_src, select_src[:, QN:]], axis=1)

    v = (src.reshape(B * N, D) @ p['value_proj_w'].T + p['value_proj_b'])
    vt = v.reshape(B * N * 2, 128)

    query = jnp.pad(select_src + select_pos, ((0, 0), (0, LQP - FG), (0, 0)))
    refp = jnp.pad(select_ref, ((0, 0), (0, LQP - FG), (0, 128 - 7)))
    idxs, wgts = _idx_wgt(query, refp, p)
    bag = _sc_bag(vt, idxs.reshape(TOTQ, NPTS), wgts.reshape(TOTQ * NPTS))
    bag = bag.reshape(B, LQP, D)

    selp = jnp.pad(select_src, ((0, 0), (0, LQP - FG), (0, 0)))
    y = _tail(selp, bag, p)[:, :FG]

    bidx = jnp.arange(Bq)[:, None]
    return src.at[bidx, indices].set(y)


# speedup vs baseline: 7.3411x; 1.1118x over previous
import functools
import math

import jax
import jax.numpy as jnp
import numpy as np
from jax import lax
from jax.experimental import pallas as pl
from jax.experimental.pallas import tpu as pltpu
from jax.experimental.pallas import tpu_sc as plsc

B = 2
N = 35344
D = 256
NH = 8
NL = 1
DFF = 512
KS = 5
NP_ = KS * KS
KEEP = 0.2
HGRID = 188
WGRID = 188
HSIZE = 188.0
DH = D // NH
FG = math.ceil(N * KEEP)
QN = math.ceil(FG * KEEP)

LQP = 7168
NPTS = 4 * NP_
TOTQ = B * LQP * NH
NTILES = 32
QPT = TOTQ // NTILES
CQ = 4
NCHUNK = QPT // CQ


def _kern_pts():
    start = -(KS - 1) / 2
    end = (KS - 1) / 2
    idx = np.linspace(start, end, KS)
    i, j = np.meshgrid(idx, idx, indexing='ij')
    kern = np.stack([j, i], axis=-1).reshape(-1, 2) / KS
    return kern.astype(np.float32)


def _kxy(shape_like):
    pi = jax.lax.broadcasted_iota(jnp.int32, (1, NP_), 1)
    kx = ((pi % KS) - (KS - 1) // 2).astype(jnp.float32) / KS
    ky = ((pi // KS) - (KS - 1) // 2).astype(jnp.float32) / KS
    return kx, ky


def _ln(x, g, b, eps=1e-5):
    mu = x.mean(-1, keepdims=True)
    var = ((x - mu) ** 2).mean(-1, keepdims=True)
    return (x - mu) / jnp.sqrt(var + eps) * g + b


def _mha_small(q, k, v, p):
    wq, wk, wv = jnp.split(p['in_proj_w'], 3, axis=0)
    bq, bk, bv = jnp.split(p['in_proj_b'], 3, axis=0)
    Bq, L, _ = q.shape

    def proj(x, w, bb):
        return (x @ w.T + bb).reshape(Bq, -1, NH, DH).transpose(0, 2, 1, 3)

    qh = proj(q, wq, bq)
    kh = proj(k, wk, bk)
    vh = proj(v, wv, bv)
    attn = jax.nn.softmax(qh @ kh.transpose(0, 1, 3, 2) / np.sqrt(DH), axis=-1)
    out = (attn @ vh).transpose(0, 2, 1, 3).reshape(Bq, L, D)
    return out @ p['mha_out_w'].T + p['mha_out_b']


def _gat(x, idx):
    ib = jnp.broadcast_to(idx[..., None], idx.shape + (x.shape[-1],))
    return jnp.take_along_axis(x, ib, axis=1)


RB = 32


def _idxwgt_body(q_ref, ref_ref, wb_ref, bb_ref, wa_ref, ba_ref,
                 wsx_ref, bsx_ref, wsy_ref, bsy_ref, idx_ref, wgt_ref):
    b = pl.program_id(0)
    q = q_ref[0]
    ob = jnp.dot(q, wb_ref[...], preferred_element_type=jnp.float32) + bb_ref[...]
    al = jnp.dot(q, wa_ref[...], preferred_element_type=jnp.float32) + ba_ref[...]
    spx = jnp.dot(q, wsx_ref[...], preferred_element_type=jnp.float32) + bsx_ref[...]
    spy = jnp.dot(q, wsy_ref[...], preferred_element_type=jnp.float32) + bsy_ref[...]
    refw = ref_ref[0]

    rcx, rcy = refw[:, 0:1], refw[:, 1:2]
    rw, rh, ang = refw[:, 3:4], refw[:, 4:5], refw[:, 6:7]
    ca = jnp.cos(ang)
    sa = jnp.sin(ang)
    kx, ky = _kxy(None)
    boff = b * (N * 2)

    idx_parts = []
    wgt_parts = []
    for h in range(NH):
        alh = al[:, h * NP_:(h + 1) * NP_]
        alh = alh - jnp.max(alh, axis=1, keepdims=True)
        ale = jnp.exp(alh)
        aw = ale / jnp.sum(ale, axis=1, keepdims=True)

        obx = ob[:, h * 4 + 0:h * 4 + 1]
        oby = ob[:, h * 4 + 1:h * 4 + 2]
        obw = ob[:, h * 4 + 2:h * 4 + 3]
        obh = ob[:, h * 4 + 3:h * 4 + 4]
        bx = rcx + obx / 8.0 * rw
        by = rcy + oby / 8.0 * rh
        bw = rw + obw / 8.0 * rw
        bh = rh + obh / 8.0 * rh
        sw = jnp.maximum(bw, 0.0)
        sh = jnp.maximum(bh, 0.0)
        fx = kx * sw
        fy = ky * sh
        gx = bx + fx * ca - fy * sa + spx[:, h * NP_:(h + 1) * NP_] / HSIZE
        gy = by + fx * sa + fy * ca + spy[:, h * NP_:(h + 1) * NP_] / HSIZE

        ix = gx * WGRID - 0.5
        iy = gy * HGRID - 0.5
        x0 = jnp.floor(ix)
        y0 = jnp.floor(iy)
        wx1 = ix - x0
        wx0 = 1.0 - wx1
        wy1 = iy - y0
        wy0 = 1.0 - wy1

        def corner(xi, yi, wx, wy):
            inb = ((xi >= 0.0) & (xi <= WGRID - 1.0)
                   & (yi >= 0.0) & (yi <= HGRID - 1.0))
            xc = jnp.clip(xi, 0.0, WGRID - 1.0).astype(jnp.int32)
            yc = jnp.clip(yi, 0.0, HGRID - 1.0).astype(jnp.int32)
            cell = yc * WGRID + xc
            fid = boff + cell * 2 + (h // 4)
            w = aw * wx * wy * inb.astype(jnp.float32)
            return fid, w

        i00, w00 = corner(x0, y0, wx0, wy0)
        i10, w10 = corner(x0 + 1.0, y0, wx1, wy0)
        i01, w01 = corner(x0, y0 + 1.0, wx0, wy1)
        i11, w11 = corner(x0 + 1.0, y0 + 1.0, wx1, wy1)
        idx_parts += [i00, i10, i01, i11]
        wgt_parts += [w00, w10, w01, w11]
    idx_ref[0] = jnp.concatenate(idx_parts, axis=1)
    wgt_ref[0] = jnp.concatenate(wgt_parts, axis=1)


def _idx_wgt(query, refp, p):
    grid = (B, LQP // RB)
    wsamp = p['samp_off_w'].reshape(NH * NP_, 2, D)
    bsamp = p['samp_off_b'].reshape(NH * NP_, 2)
    out = pl.pallas_call(
        _idxwgt_body,
        out_shape=(
            jax.ShapeDtypeStruct((B, LQP, NH * NPTS), jnp.int32),
            jax.ShapeDtypeStruct((B, LQP, NH * NPTS), jnp.float32),
        ),
        grid=grid,
        in_specs=[
            pl.BlockSpec((1, RB, D), lambda b, i: (b, i, 0)),
            pl.BlockSpec((1, RB, 128), lambda b, i: (b, i, 0)),
            pl.BlockSpec((D, NH * 4), lambda b, i: (0, 0)),
            pl.BlockSpec((NH * 4,), lambda b, i: (0,)),
            pl.BlockSpec((D, NH * NP_), lambda b, i: (0, 0)),
            pl.BlockSpec((NH * NP_,), lambda b, i: (0,)),
            pl.BlockSpec((D, NH * NP_), lambda b, i: (0, 0)),
            pl.BlockSpec((NH * NP_,), lambda b, i: (0,)),
            pl.BlockSpec((D, NH * NP_), lambda b, i: (0, 0)),
            pl.BlockSpec((NH * NP_,), lambda b, i: (0,)),
        ],
        out_specs=(
            pl.BlockSpec((1, RB, NH * NPTS), lambda b, i: (b, i, 0)),
            pl.BlockSpec((1, RB, NH * NPTS), lambda b, i: (b, i, 0)),
        ),
    )(query, refp,
      p['linear_box_w'].T, p['linear_box_b'],
      p['attn_w_w'].T, p['attn_w_b'],
      wsamp[:, 0, :].T, bsamp[:, 0],
      wsamp[:, 1, :].T, bsamp[:, 1])
    return out


def _sc_bag(vt, idxs, wgts):
    mesh = plsc.VectorSubcoreMesh(core_axis_name="c", subcore_axis_name="s")

    @functools.partial(
        pl.kernel, mesh=mesh,
        compiler_params=pltpu.CompilerParams(needs_layout_passes=False),
        out_type=jax.ShapeDtypeStruct((TOTQ, DH), jnp.float32),
        scratch_types=[
            pltpu.VMEM((2, CQ, NPTS), jnp.int32),
            pltpu.VMEM((2 * CQ * NPTS,), jnp.float32),
            pltpu.VMEM((2, CQ * NPTS, 128), jnp.float32),
            pltpu.VMEM((CQ, DH), jnp.float32),
            pltpu.SemaphoreType.DMA,
            pltpu.SemaphoreType.DMA,
        ],
    )
    def bag(vt_hbm, idx_hbm, wgt_hbm, out_hbm, idx_v, wgt_v, rows_v, out_v,
            gsem0, gsem1):
        wid = lax.axis_index("s") * 2 + lax.axis_index("c")
        base = wid * QPT
        sems = (gsem0, gsem1)

        def fire(s, off):
            pltpu.sync_copy(idx_hbm.at[pl.ds(off, CQ)], idx_v.at[s])
            pltpu.sync_copy(wgt_hbm.at[pl.ds(off * NPTS, CQ * NPTS)],
                            wgt_v.at[pl.ds(s * CQ * NPTS, CQ * NPTS)])
            for q in range(CQ):
                pltpu.async_copy(
                    vt_hbm.at[idx_v.at[s, q]],
                    rows_v.at[s, pl.ds(q * NPTS, NPTS)], sems[s])

        def wait_fired(s):
            for q in range(CQ):
                pltpu.make_async_copy(
                    vt_hbm.at[idx_v.at[s, q]],
                    rows_v.at[s, pl.ds(q * NPTS, NPTS)], sems[s]).wait()

        def compute(s, off):
            def qbody(q, _):
                rb = q * NPTS
                hoff = q * DH
                qv = jnp.full((16,), s * CQ * NPTS + q * NPTS, jnp.int32)
                acc0 = jnp.zeros((16,), jnp.float32)
                acc1 = jnp.zeros((16,), jnp.float32)
                for j in range(NPTS):
                    wv = plsc.load_gather(wgt_v, [qv + j])
                    acc0 = acc0 + wv * rows_v[s, rb + j, pl.ds(hoff, 16)]
                    acc1 = acc1 + wv * rows_v[s, rb + j, pl.ds(hoff + 16, 16)]
                out_v[q, pl.ds(0, 16)] = acc0
                out_v[q, pl.ds(16, 16)] = acc1
                return 0

            lax.fori_loop(0, CQ, qbody, 0)
            pltpu.sync_copy(out_v, out_hbm.at[pl.ds(off, CQ)])

        fire(0, base)

        def pair(i, _):
            g0 = 2 * i
            fire(1, base + (g0 + 1) * CQ)
            wait_fired(0)
            compute(0, base + g0 * CQ)

            @pl.when(g0 + 2 < NCHUNK)
            def _():
                fire(0, base + (g0 + 2) * CQ)

            wait_fired(1)
            compute(1, base + (g0 + 1) * CQ)
            return 0

        lax.fori_loop(0, NCHUNK // 2, pair, 0)

    return bag(vt, idxs, wgts)


RB2 = 512


def _tail_body(sel_ref, bag_ref, wo_ref, bo_ref, g1_ref, b1_ref,
               w1_ref, bb1_ref, w2_ref, bb2_ref, g2_ref, b2_ref, o_ref):
    bagp = jnp.dot(bag_ref[0], wo_ref[...],
                   preferred_element_type=jnp.float32) + bo_ref[...]
    x = sel_ref[0] + bagp
    mu = x.mean(-1, keepdims=True)
    var = ((x - mu) ** 2).mean(-1, keepdims=True)
    x = (x - mu) / jnp.sqrt(var + 1e-5) * g1_ref[...] + b1_ref[...]
    h = jnp.maximum(
        jnp.dot(x, w1_ref[...], preferred_element_type=jnp.float32)
        + bb1_ref[...], 0.0)
    y = x + jnp.dot(h, w2_ref[...], preferred_element_type=jnp.float32) \
        + bb2_ref[...]
    mu = y.mean(-1, keepdims=True)
    var = ((y - mu) ** 2).mean(-1, keepdims=True)
    o_ref[0] = (y - mu) / jnp.sqrt(var + 1e-5) * g2_ref[...] + b2_ref[...]


def _tail(sel, bag, p):
    grid = (B * LQP // RB2,)
    sel = sel.reshape(B * LQP // RB2, RB2, D)
    bag = bag.reshape(B * LQP // RB2, RB2, D)
    vec = lambda: pl.BlockSpec((D,), lambda i: (0,))
    mat = lambda s: pl.BlockSpec(s, lambda i: (0, 0))
    out = pl.pallas_call(
        _tail_body,
        out_shape=jax.ShapeDtypeStruct((B * LQP // RB2, RB2, D), jnp.float32),
        grid=grid,
        in_specs=[
            pl.BlockSpec((1, RB2, D), lambda i: (i, 0, 0)),
            pl.BlockSpec((1, RB2, D), lambda i: (i, 0, 0)),
            mat((D, D)), vec(), vec(), vec(),
            mat((D, DFF)), pl.BlockSpec((DFF,), lambda i: (0,)),
            mat((DFF, D)), vec(), vec(), vec(),
        ],
        out_specs=pl.BlockSpec((1, RB2, D), lambda i: (i, 0, 0)),
    )(sel, bag, p['out_proj_w'].T, p['out_proj_b'],
      p['norm1_g'], p['norm1_b'],
      p['lin1_w'].T, p['lin1_b'], p['lin2_w'].T, p['lin2_b'],
      p['norm2_g'], p['norm2_b'])
    return out.reshape(B, LQP, D)


def kernel(src, pos, src_shape, src_start_idx, ref_windows, score_mask,
           params):
    p = params
    Bq = src.shape[0]
    sel_score, indices = jax.lax.top_k(score_mask, FG)
    select_src = _gat(src, indices)
    select_pos = _gat(pos, indices)
    select_ref = _gat(ref_windows, indices)
    query_src = select_src[:, :QN]
    query_pos = select_pos[:, :QN]
    q = query_src + query_pos
    q2 = _mha_small(q, q, query_src, p)
    query_src = _ln(query_src + q2, p['query_norm_g'], p['query_norm_b'])
    select_src = jnp.concatenate([query_src, select_src[:, QN:]], axis=1)

    v = (src.reshape(B * N, D) @ p['value_proj_w'].T + p['value_proj_b'])
    vt = v.reshape(B * N * 2, 128)

    query = jnp.pad(select_src + select_pos, ((0, 0), (0, LQP - FG), (0, 0)))
    refp = jnp.pad(select_ref, ((0, 0), (0, LQP - FG), (0, 128 - 7)))
    idxs, wgts = _idx_wgt(query, refp, p)
    bag = _sc_bag(vt, idxs.reshape(TOTQ, NPTS), wgts.reshape(TOTQ * NPTS))
    bag = bag.reshape(B, LQP, D)

    selp = jnp.pad(select_src, ((0, 0), (0, LQP - FG), (0, 0)))
    y = _tail(selp, bag, p)[:, :FG]

    bidx = jnp.arange(Bq)[:, None]
    return src.at[bidx, indices].set(y)


# speedup vs baseline: 12.4814x; 1.7002x over previous
import functools
import math

import jax
import jax.numpy as jnp
import numpy as np
from jax import lax
from jax.experimental import pallas as pl
from jax.experimental.pallas import tpu as pltpu
from jax.experimental.pallas import tpu_sc as plsc

B = 2
N = 35344
D = 256
NH = 8
NL = 1
DFF = 512
KS = 5
NP_ = KS * KS
KEEP = 0.2
HGRID = 188
WGRID = 188
HSIZE = 188.0
DH = D // NH
FG = math.ceil(N * KEEP)
QN = math.ceil(FG * KEEP)

LQP = 7168
NPTS = 4 * NP_
PG = HGRID + 2
TOTQ = B * LQP * NH
NTILES = 32
QPT = TOTQ // NTILES
CQ = 16
NCHUNK = QPT // CQ


def _kern_pts():
    start = -(KS - 1) / 2
    end = (KS - 1) / 2
    idx = np.linspace(start, end, KS)
    i, j = np.meshgrid(idx, idx, indexing='ij')
    kern = np.stack([j, i], axis=-1).reshape(-1, 2) / KS
    return kern.astype(np.float32)


def _kxy(shape_like):
    pi = jax.lax.broadcasted_iota(jnp.int32, (1, NP_), 1)
    kx = ((pi % KS) - (KS - 1) // 2).astype(jnp.float32) / KS
    ky = ((pi // KS) - (KS - 1) // 2).astype(jnp.float32) / KS
    return kx, ky


def _ln(x, g, b, eps=1e-5):
    mu = x.mean(-1, keepdims=True)
    var = ((x - mu) ** 2).mean(-1, keepdims=True)
    return (x - mu) / jnp.sqrt(var + eps) * g + b


def _mha_small(q, k, v, p):
    wq, wk, wv = jnp.split(p['in_proj_w'], 3, axis=0)
    bq, bk, bv = jnp.split(p['in_proj_b'], 3, axis=0)
    Bq, L, _ = q.shape

    def proj(x, w, bb):
        return (x @ w.T + bb).reshape(Bq, -1, NH, DH).transpose(0, 2, 1, 3)

    qh = proj(q, wq, bq)
    kh = proj(k, wk, bk)
    vh = proj(v, wv, bv)
    attn = jax.nn.softmax(qh @ kh.transpose(0, 1, 3, 2) / np.sqrt(DH), axis=-1)
    out = (attn @ vh).transpose(0, 2, 1, 3).reshape(Bq, L, D)
    return out @ p['mha_out_w'].T + p['mha_out_b']


def _gat(x, idx):
    ib = jnp.broadcast_to(idx[..., None], idx.shape + (x.shape[-1],))
    return jnp.take_along_axis(x, ib, axis=1)


RB = 32


def _idxwgt_body(q_ref, ref_ref, wb_ref, bb_ref, wa_ref, ba_ref,
                 wsx_ref, bsx_ref, wsy_ref, bsy_ref, idx_ref, wgt_ref):
    b = pl.program_id(0)
    q = q_ref[0]
    ob = jnp.dot(q, wb_ref[...], preferred_element_type=jnp.float32) + bb_ref[...]
    al = jnp.dot(q, wa_ref[...], preferred_element_type=jnp.float32) + ba_ref[...]
    spx = jnp.dot(q, wsx_ref[...], preferred_element_type=jnp.float32) + bsx_ref[...]
    spy = jnp.dot(q, wsy_ref[...], preferred_element_type=jnp.float32) + bsy_ref[...]
    refw = ref_ref[0]

    rcx, rcy = refw[:, 0:1], refw[:, 1:2]
    rw, rh, ang = refw[:, 3:4], refw[:, 4:5], refw[:, 6:7]
    ca = jnp.cos(ang)
    sa = jnp.sin(ang)
    kx, ky = _kxy(None)
    boff = b * (PG * PG * NH)

    idx_parts = []
    wgt_parts = []
    for h in range(NH):
        alh = al[:, h * NP_:(h + 1) * NP_]
        alh = alh - jnp.max(alh, axis=1, keepdims=True)
        ale = jnp.exp(alh)
        aw = ale / jnp.sum(ale, axis=1, keepdims=True)

        obx = ob[:, h * 4 + 0:h * 4 + 1]
        oby = ob[:, h * 4 + 1:h * 4 + 2]
        obw = ob[:, h * 4 + 2:h * 4 + 3]
        obh = ob[:, h * 4 + 3:h * 4 + 4]
        bx = rcx + obx / 8.0 * rw
        by = rcy + oby / 8.0 * rh
        bw = rw + obw / 8.0 * rw
        bh = rh + obh / 8.0 * rh
        sw = jnp.maximum(bw, 0.0)
        sh = jnp.maximum(bh, 0.0)
        fx = kx * sw
        fy = ky * sh
        gx = bx + fx * ca - fy * sa + spx[:, h * NP_:(h + 1) * NP_] / HSIZE
        gy = by + fx * sa + fy * ca + spy[:, h * NP_:(h + 1) * NP_] / HSIZE

        ix = gx * WGRID - 0.5
        iy = gy * HGRID - 0.5
        x0 = jnp.floor(ix)
        y0 = jnp.floor(iy)
        wx1 = ix - x0
        wx0 = 1.0 - wx1
        wy1 = iy - y0
        wy0 = 1.0 - wy1

        x0p = jnp.clip(x0 + 1.0, 0.0, PG - 1.0).astype(jnp.int32)
        y0p = jnp.clip(y0 + 1.0, 0.0, PG - 1.0).astype(jnp.int32)
        pidx = boff + (y0p * PG + x0p) * NH + h

        def cwgt(xi, yi, wx, wy):
            inb = ((xi >= 0.0) & (xi <= WGRID - 1.0)
                   & (yi >= 0.0) & (yi <= HGRID - 1.0))
            return aw * wx * wy * inb.astype(jnp.float32)

        idx_parts += [pidx]
        wgt_parts += [cwgt(x0, y0, wx0, wy0),
                      cwgt(x0 + 1.0, y0, wx1, wy0),
                      cwgt(x0, y0 + 1.0, wx0, wy1),
                      cwgt(x0 + 1.0, y0 + 1.0, wx1, wy1)]
    idx_ref[0] = jnp.concatenate(idx_parts, axis=1)
    wgt_ref[0] = jnp.concatenate(wgt_parts, axis=1)


def _idx_wgt(query, refp, p):
    grid = (B, LQP // RB)
    wsamp = p['samp_off_w'].reshape(NH * NP_, 2, D)
    bsamp = p['samp_off_b'].reshape(NH * NP_, 2)
    out = pl.pallas_call(
        _idxwgt_body,
        out_shape=(
            jax.ShapeDtypeStruct((B, LQP, NH * NP_), jnp.int32),
            jax.ShapeDtypeStruct((B, LQP, NH * NPTS), jnp.float32),
        ),
        grid=grid,
        in_specs=[
            pl.BlockSpec((1, RB, D), lambda b, i: (b, i, 0)),
            pl.BlockSpec((1, RB, 128), lambda b, i: (b, i, 0)),
            pl.BlockSpec((D, NH * 4), lambda b, i: (0, 0)),
            pl.BlockSpec((NH * 4,), lambda b, i: (0,)),
            pl.BlockSpec((D, NH * NP_), lambda b, i: (0, 0)),
            pl.BlockSpec((NH * NP_,), lambda b, i: (0,)),
            pl.BlockSpec((D, NH * NP_), lambda b, i: (0, 0)),
            pl.BlockSpec((NH * NP_,), lambda b, i: (0,)),
            pl.BlockSpec((D, NH * NP_), lambda b, i: (0, 0)),
            pl.BlockSpec((NH * NP_,), lambda b, i: (0,)),
        ],
        out_specs=(
            pl.BlockSpec((1, RB, NH * NP_), lambda b, i: (b, i, 0)),
            pl.BlockSpec((1, RB, NH * NPTS), lambda b, i: (b, i, 0)),
        ),
    )(query, refp,
      p['linear_box_w'].T, p['linear_box_b'],
      p['attn_w_w'].T, p['attn_w_b'],
      wsamp[:, 0, :].T, bsamp[:, 0],
      wsamp[:, 1, :].T, bsamp[:, 1])
    return out


def _sc_bag(vt, idxs, wgts):
    mesh = plsc.VectorSubcoreMesh(core_axis_name="c", subcore_axis_name="s")

    @functools.partial(
        pl.kernel, mesh=mesh,
        compiler_params=pltpu.CompilerParams(needs_layout_passes=False),
        out_type=jax.ShapeDtypeStruct((TOTQ, DH), jnp.float32),
        scratch_types=[
            pltpu.VMEM((2, CQ, NP_), jnp.int32),
            pltpu.VMEM((2 * CQ * NPTS,), jnp.float32),
            pltpu.VMEM((2, CQ * NP_, 128), jnp.float32),
            pltpu.VMEM((CQ, DH), jnp.float32),
            pltpu.SemaphoreType.DMA,
            pltpu.SemaphoreType.DMA,
        ],
    )
    def bag(vt_hbm, idx_hbm, wgt_hbm, out_hbm, idx_v, wgt_v, rows_v, out_v,
            gsem0, gsem1):
        wid = lax.axis_index("s") * 2 + lax.axis_index("c")
        base = wid * QPT
        sems = (gsem0, gsem1)

        def fire(s, off):
            pltpu.sync_copy(idx_hbm.at[pl.ds(off, CQ)], idx_v.at[s])
            pltpu.sync_copy(wgt_hbm.at[pl.ds(off * NPTS, CQ * NPTS)],
                            wgt_v.at[pl.ds(s * CQ * NPTS, CQ * NPTS)])
            for q in range(CQ):
                pltpu.async_copy(
                    vt_hbm.at[idx_v.at[s, q]],
                    rows_v.at[s, pl.ds(q * NP_, NP_)], sems[s])

        def wait_fired(s):
            for q in range(CQ):
                pltpu.make_async_copy(
                    vt_hbm.at[idx_v.at[s, q]],
                    rows_v.at[s, pl.ds(q * NP_, NP_)], sems[s]).wait()

        def compute(s, off):
            def qbody(q, _):
                rb = q * NP_
                qv = jnp.full((16,), s * CQ * NPTS + q * NPTS, jnp.int32)
                acc0 = jnp.zeros((16,), jnp.float32)
                acc1 = jnp.zeros((16,), jnp.float32)
                for p in range(NP_):
                    for c in range(4):
                        wv = plsc.load_gather(wgt_v, [qv + (c * NP_ + p)])
                        co = c * DH
                        acc0 = acc0 + wv * rows_v[s, rb + p, pl.ds(co, 16)]
                        acc1 = acc1 + wv * rows_v[s, rb + p,
                                                  pl.ds(co + 16, 16)]
                out_v[q, pl.ds(0, 16)] = acc0
                out_v[q, pl.ds(16, 16)] = acc1
                return 0

            lax.fori_loop(0, CQ, qbody, 0)
            pltpu.sync_copy(out_v, out_hbm.at[pl.ds(off, CQ)])

        fire(0, base)

        def pair(i, _):
            g0 = 2 * i
            fire(1, base + (g0 + 1) * CQ)
            wait_fired(0)
            compute(0, base + g0 * CQ)

            @pl.when(g0 + 2 < NCHUNK)
            def _():
                fire(0, base + (g0 + 2) * CQ)

            wait_fired(1)
            compute(1, base + (g0 + 1) * CQ)
            return 0

        lax.fori_loop(0, NCHUNK // 2, pair, 0)

    return bag(vt, idxs, wgts)


RB2 = 512


def _tail_body(sel_ref, bag_ref, wo_ref, bo_ref, g1_ref, b1_ref,
               w1_ref, bb1_ref, w2_ref, bb2_ref, g2_ref, b2_ref, o_ref):
    bagp = jnp.dot(bag_ref[0], wo_ref[...],
                   preferred_element_type=jnp.float32) + bo_ref[...]
    x = sel_ref[0] + bagp
    mu = x.mean(-1, keepdims=True)
    var = ((x - mu) ** 2).mean(-1, keepdims=True)
    x = (x - mu) / jnp.sqrt(var + 1e-5) * g1_ref[...] + b1_ref[...]
    h = jnp.maximum(
        jnp.dot(x, w1_ref[...], preferred_element_type=jnp.float32)
        + bb1_ref[...], 0.0)
    y = x + jnp.dot(h, w2_ref[...], preferred_element_type=jnp.float32) \
        + bb2_ref[...]
    mu = y.mean(-1, keepdims=True)
    var = ((y - mu) ** 2).mean(-1, keepdims=True)
    o_ref[0] = (y - mu) / jnp.sqrt(var + 1e-5) * g2_ref[...] + b2_ref[...]


def _tail(sel, bag, p):
    grid = (B * LQP // RB2,)
    sel = sel.reshape(B * LQP // RB2, RB2, D)
    bag = bag.reshape(B * LQP // RB2, RB2, D)
    vec = lambda: pl.BlockSpec((D,), lambda i: (0,))
    mat = lambda s: pl.BlockSpec(s, lambda i: (0, 0))
    out = pl.pallas_call(
        _tail_body,
        out_shape=jax.ShapeDtypeStruct((B * LQP // RB2, RB2, D), jnp.float32),
        grid=grid,
        in_specs=[
            pl.BlockSpec((1, RB2, D), lambda i: (i, 0, 0)),
            pl.BlockSpec((1, RB2, D), lambda i: (i, 0, 0)),
            mat((D, D)), vec(), vec(), vec(),
            mat((D, DFF)), pl.BlockSpec((DFF,), lambda i: (0,)),
            mat((DFF, D)), vec(), vec(), vec(),
        ],
        out_specs=pl.BlockSpec((1, RB2, D), lambda i: (i, 0, 0)),
    )(sel, bag, p['out_proj_w'].T, p['out_proj_b'],
      p['norm1_g'], p['norm1_b'],
      p['lin1_w'].T, p['lin1_b'], p['lin2_w'].T, p['lin2_b'],
      p['norm2_g'], p['norm2_b'])
    return out.reshape(B, LQP, D)


def kernel(src, pos, src_shape, src_start_idx, ref_windows, score_mask,
           params):
    p = params
    Bq = src.shape[0]
    sel_score, indices = jax.lax.top_k(score_mask, FG)
    select_src = _gat(src, indices)
    select_pos = _gat(pos, indices)
    select_ref = _gat(ref_windows, indices)
    query_src = select_src[:, :QN]
    query_pos = select_pos[:, :QN]
    q = query_src + query_pos
    q2 = _mha_small(q, q, query_src, p)
    query_src = _ln(query_src + q2, p['query_norm_g'], p['query_norm_b'])
    select_src = jnp.concatenate([query_src, select_src[:, QN:]], axis=1)

    v = (src.reshape(B * N, D) @ p['value_proj_w'].T + p['value_proj_b'])
    vg = jnp.pad(v.reshape(B, HGRID, WGRID, NH, DH),
                 ((0, 0), (1, 2), (1, 2), (0, 0), (0, 0)))
    vn = jnp.stack([vg[:, :-1, :-1], vg[:, :-1, 1:],
                    vg[:, 1:, :-1], vg[:, 1:, 1:]], axis=4)
    vt = vn.reshape(B * PG * PG * NH, 4 * DH)

    query = jnp.pad(select_src + select_pos, ((0, 0), (0, LQP - FG), (0, 0)))
    refp = jnp.pad(select_ref, ((0, 0), (0, LQP - FG), (0, 128 - 7)))
    idxs, wgts = _idx_wgt(query, refp, p)
    bag = _sc_bag(vt, idxs.reshape(TOTQ, NP_), wgts.reshape(TOTQ * NPTS))
    bag = bag.reshape(B, LQP, D)

    selp = jnp.pad(select_src, ((0, 0), (0, LQP - FG), (0, 0)))
    y = _tail(selp, bag, p)[:, :FG]

    bidx = jnp.arange(Bq)[:, None]
    return src.at[bidx, indices].set(y)


# speedup vs baseline: 12.8182x; 1.0270x over previous
import functools
import math

import jax
import jax.numpy as jnp
import numpy as np
from jax import lax
from jax.experimental import pallas as pl
from jax.experimental.pallas import tpu as pltpu
from jax.experimental.pallas import tpu_sc as plsc

B = 2
N = 35344
D = 256
NH = 8
NL = 1
DFF = 512
KS = 5
NP_ = KS * KS
KEEP = 0.2
HGRID = 188
WGRID = 188
HSIZE = 188.0
DH = D // NH
FG = math.ceil(N * KEEP)
QN = math.ceil(FG * KEEP)

LQP = 7168
NPTS = 4 * NP_
PG = HGRID + 2
TOTQ = B * LQP * NH
NTILES = 32
QPT = TOTQ // NTILES
CQ = 16
NCHUNK = QPT // CQ


def _kern_pts():
    start = -(KS - 1) / 2
    end = (KS - 1) / 2
    idx = np.linspace(start, end, KS)
    i, j = np.meshgrid(idx, idx, indexing='ij')
    kern = np.stack([j, i], axis=-1).reshape(-1, 2) / KS
    return kern.astype(np.float32)


def _kxy(shape_like):
    pi = jax.lax.broadcasted_iota(jnp.int32, (1, NP_), 1)
    kx = ((pi % KS) - (KS - 1) // 2).astype(jnp.float32) / KS
    ky = ((pi // KS) - (KS - 1) // 2).astype(jnp.float32) / KS
    return kx, ky


def _ln(x, g, b, eps=1e-5):
    mu = x.mean(-1, keepdims=True)
    var = ((x - mu) ** 2).mean(-1, keepdims=True)
    return (x - mu) / jnp.sqrt(var + eps) * g + b


def _mha_small(q, k, v, p):
    wq, wk, wv = jnp.split(p['in_proj_w'], 3, axis=0)
    bq, bk, bv = jnp.split(p['in_proj_b'], 3, axis=0)
    Bq, L, _ = q.shape

    def proj(x, w, bb):
        return (x @ w.T + bb).reshape(Bq, -1, NH, DH).transpose(0, 2, 1, 3)

    qh = proj(q, wq, bq)
    kh = proj(k, wk, bk)
    vh = proj(v, wv, bv)
    attn = jax.nn.softmax(qh @ kh.transpose(0, 1, 3, 2) / np.sqrt(DH), axis=-1)
    out = (attn @ vh).transpose(0, 2, 1, 3).reshape(Bq, L, D)
    return out @ p['mha_out_w'].T + p['mha_out_b']


def _gat(x, idx):
    ib = jnp.broadcast_to(idx[..., None], idx.shape + (x.shape[-1],))
    return jnp.take_along_axis(x, ib, axis=1)


RB = 32


def _idxwgt_body(q_ref, ref_ref, wb_ref, bb_ref, wa_ref, ba_ref,
                 wsx_ref, bsx_ref, wsy_ref, bsy_ref, idx_ref, wgt_ref):
    b = pl.program_id(0)
    q = q_ref[0]
    ob = jnp.dot(q, wb_ref[...], preferred_element_type=jnp.float32) + bb_ref[...]
    al = jnp.dot(q, wa_ref[...], preferred_element_type=jnp.float32) + ba_ref[...]
    spx = jnp.dot(q, wsx_ref[...], preferred_element_type=jnp.float32) + bsx_ref[...]
    spy = jnp.dot(q, wsy_ref[...], preferred_element_type=jnp.float32) + bsy_ref[...]
    refw = ref_ref[0]

    rcx, rcy = refw[:, 0:1], refw[:, 1:2]
    rw, rh, ang = refw[:, 3:4], refw[:, 4:5], refw[:, 6:7]
    ca = jnp.cos(ang)
    sa = jnp.sin(ang)
    kx, ky = _kxy(None)
    boff = b * (PG * PG * NH)

    idx_parts = []
    wgt_parts = []
    for h in range(NH):
        alh = al[:, h * NP_:(h + 1) * NP_]
        alh = alh - jnp.max(alh, axis=1, keepdims=True)
        ale = jnp.exp(alh)
        aw = ale / jnp.sum(ale, axis=1, keepdims=True)

        obx = ob[:, h * 4 + 0:h * 4 + 1]
        oby = ob[:, h * 4 + 1:h * 4 + 2]
        obw = ob[:, h * 4 + 2:h * 4 + 3]
        obh = ob[:, h * 4 + 3:h * 4 + 4]
        bx = rcx + obx / 8.0 * rw
        by = rcy + oby / 8.0 * rh
        bw = rw + obw / 8.0 * rw
        bh = rh + obh / 8.0 * rh
        sw = jnp.maximum(bw, 0.0)
        sh = jnp.maximum(bh, 0.0)
        fx = kx * sw
        fy = ky * sh
        gx = bx + fx * ca - fy * sa + spx[:, h * NP_:(h + 1) * NP_] / HSIZE
        gy = by + fx * sa + fy * ca + spy[:, h * NP_:(h + 1) * NP_] / HSIZE

        ix = gx * WGRID - 0.5
        iy = gy * HGRID - 0.5
        x0 = jnp.floor(ix)
        y0 = jnp.floor(iy)
        wx1 = ix - x0
        wx0 = 1.0 - wx1
        wy1 = iy - y0
        wy0 = 1.0 - wy1

        x0p = jnp.clip(x0 + 1.0, 0.0, PG - 1.0).astype(jnp.int32)
        y0p = jnp.clip(y0 + 1.0, 0.0, PG - 1.0).astype(jnp.int32)
        pidx = boff + (y0p * PG + x0p) * NH + h

        def cwgt(xi, yi, wx, wy):
            inb = ((xi >= 0.0) & (xi <= WGRID - 1.0)
                   & (yi >= 0.0) & (yi <= HGRID - 1.0))
            return aw * wx * wy * inb.astype(jnp.float32)

        idx_parts += [pidx]
        wgt_parts += [cwgt(x0, y0, wx0, wy0),
                      cwgt(x0 + 1.0, y0, wx1, wy0),
                      cwgt(x0, y0 + 1.0, wx0, wy1),
                      cwgt(x0 + 1.0, y0 + 1.0, wx1, wy1)]
    idx_ref[0] = jnp.concatenate(idx_parts, axis=1)
    wgt_ref[0] = jnp.concatenate(wgt_parts, axis=1)


def _idx_wgt(query, refp, p):
    grid = (B, LQP // RB)
    wsamp = p['samp_off_w'].reshape(NH * NP_, 2, D)
    bsamp = p['samp_off_b'].reshape(NH * NP_, 2)
    out = pl.pallas_call(
        _idxwgt_body,
        out_shape=(
            jax.ShapeDtypeStruct((B, LQP, NH * NP_), jnp.int32),
            jax.ShapeDtypeStruct((B, LQP, NH * NPTS), jnp.float32),
        ),
        grid=grid,
        in_specs=[
            pl.BlockSpec((1, RB, D), lambda b, i: (b, i, 0)),
            pl.BlockSpec((1, RB, 128), lambda b, i: (b, i, 0)),
            pl.BlockSpec((D, NH * 4), lambda b, i: (0, 0)),
            pl.BlockSpec((NH * 4,), lambda b, i: (0,)),
            pl.BlockSpec((D, NH * NP_), lambda b, i: (0, 0)),
            pl.BlockSpec((NH * NP_,), lambda b, i: (0,)),
            pl.BlockSpec((D, NH * NP_), lambda b, i: (0, 0)),
            pl.BlockSpec((NH * NP_,), lambda b, i: (0,)),
            pl.BlockSpec((D, NH * NP_), lambda b, i: (0, 0)),
            pl.BlockSpec((NH * NP_,), lambda b, i: (0,)),
        ],
        out_specs=(
            pl.BlockSpec((1, RB, NH * NP_), lambda b, i: (b, i, 0)),
            pl.BlockSpec((1, RB, NH * NPTS), lambda b, i: (b, i, 0)),
        ),
    )(query, refp,
      p['linear_box_w'].T, p['linear_box_b'],
      p['attn_w_w'].T, p['attn_w_b'],
      wsamp[:, 0, :].T, bsamp[:, 0],
      wsamp[:, 1, :].T, bsamp[:, 1])
    return out


def _sc_bag(vt, idxs, wgts):
    mesh = plsc.VectorSubcoreMesh(core_axis_name="c", subcore_axis_name="s")

    @functools.partial(
        pl.kernel, mesh=mesh,
        compiler_params=pltpu.CompilerParams(needs_layout_passes=False),
        out_type=jax.ShapeDtypeStruct((TOTQ, DH), jnp.float32),
        scratch_types=[
            pltpu.VMEM((2, CQ, NP_), jnp.int32),
            pltpu.VMEM((2 * CQ * NPTS,), jnp.float32),
            pltpu.VMEM((2, CQ * NP_, 128), jnp.float32),
            pltpu.VMEM((CQ, DH), jnp.float32),
            pltpu.SemaphoreType.DMA,
            pltpu.SemaphoreType.DMA,
        ],
    )
    def bag(vt_hbm, idx_hbm, wgt_hbm, out_hbm, idx_v, wgt_v, rows_v, out_v,
            gsem0, gsem1):
        wid = lax.axis_index("s") * 2 + lax.axis_index("c")
        base = wid * QPT
        sems = (gsem0, gsem1)

        def fire(s, off):
            pltpu.sync_copy(idx_hbm.at[pl.ds(off, CQ)], idx_v.at[s])
            pltpu.sync_copy(wgt_hbm.at[pl.ds(off * NPTS, CQ * NPTS)],
                            wgt_v.at[pl.ds(s * CQ * NPTS, CQ * NPTS)])
            for q in range(CQ):
                pltpu.async_copy(
                    vt_hbm.at[idx_v.at[s, q]],
                    rows_v.at[s, pl.ds(q * NP_, NP_)], sems[s])

        def wait_fired(s):
            for q in range(CQ):
                pltpu.make_async_copy(
                    vt_hbm.at[idx_v.at[s, q]],
                    rows_v.at[s, pl.ds(q * NP_, NP_)], sems[s]).wait()

        def compute(s, off):
            def qbody(q, _):
                rb = q * NP_
                qv = jnp.full((16,), s * CQ * NPTS + q * NPTS, jnp.int32)
                acc0 = jnp.zeros((16,), jnp.float32)
                acc1 = jnp.zeros((16,), jnp.float32)
                for p in range(NP_):
                    for c in range(4):
                        wv = plsc.load_gather(wgt_v, [qv + (c * NP_ + p)])
                        co = c * DH
                        acc0 = acc0 + wv * rows_v[s, rb + p, pl.ds(co, 16)]
                        acc1 = acc1 + wv * rows_v[s, rb + p,
                                                  pl.ds(co + 16, 16)]
                out_v[q, pl.ds(0, 16)] = acc0
                out_v[q, pl.ds(16, 16)] = acc1
                return 0

            lax.fori_loop(0, CQ, qbody, 0)
            pltpu.sync_copy(out_v, out_hbm.at[pl.ds(off, CQ)])

        fire(0, base)

        def pair(i, _):
            g0 = 2 * i
            fire(1, base + (g0 + 1) * CQ)
            wait_fired(0)
            compute(0, base + g0 * CQ)

            @pl.when(g0 + 2 < NCHUNK)
            def _():
                fire(0, base + (g0 + 2) * CQ)

            wait_fired(1)
            compute(1, base + (g0 + 1) * CQ)
            return 0

        lax.fori_loop(0, NCHUNK // 2, pair, 0)

    return bag(vt, idxs, wgts)


GR = 64
GPT = B * LQP // NTILES
GCH = GPT // GR


def _sc_select(src2, pos2, fidx):
    mesh = plsc.VectorSubcoreMesh(core_axis_name="c", subcore_axis_name="s")

    @functools.partial(
        pl.kernel, mesh=mesh,
        compiler_params=pltpu.CompilerParams(needs_layout_passes=False),
        out_type=(jax.ShapeDtypeStruct((B * LQP, D), jnp.float32),
                  jax.ShapeDtypeStruct((B * LQP, D), jnp.float32)),
        scratch_types=[
            pltpu.VMEM((GR,), jnp.int32),
            pltpu.VMEM((GR, D), jnp.float32),
            pltpu.VMEM((GR, D), jnp.float32),
            pltpu.SemaphoreType.DMA,
        ],
    )
    def sel(src_hbm, pos_hbm, fidx_hbm, osrc_hbm, opos_hbm,
            idx_v, bs_v, bp_v, sem):
        wid = lax.axis_index("s") * 2 + lax.axis_index("c")
        base = wid * GPT

        def chunk(c, _):
            off = base + c * GR
            pltpu.sync_copy(fidx_hbm.at[pl.ds(off, GR)], idx_v)
            cs = pltpu.async_copy(src_hbm.at[idx_v], bs_v, sem)
            cp = pltpu.async_copy(pos_hbm.at[idx_v], bp_v, sem)
            cs.wait()
            cp.wait()
            pltpu.sync_copy(bs_v, osrc_hbm.at[pl.ds(off, GR)])
            pltpu.sync_copy(bp_v, opos_hbm.at[pl.ds(off, GR)])
            return 0

        lax.fori_loop(0, GCH, chunk, 0)

    return sel(src2, pos2, fidx)


RB2 = 512


def _tail_body(sel_ref, bag_ref, wo_ref, bo_ref, g1_ref, b1_ref,
               w1_ref, bb1_ref, w2_ref, bb2_ref, g2_ref, b2_ref, o_ref):
    bagp = jnp.dot(bag_ref[0], wo_ref[...],
                   preferred_element_type=jnp.float32) + bo_ref[...]
    x = sel_ref[0] + bagp
    mu = x.mean(-1, keepdims=True)
    var = ((x - mu) ** 2).mean(-1, keepdims=True)
    x = (x - mu) / jnp.sqrt(var + 1e-5) * g1_ref[...] + b1_ref[...]
    h = jnp.maximum(
        jnp.dot(x, w1_ref[...], preferred_element_type=jnp.float32)
        + bb1_ref[...], 0.0)
    y = x + jnp.dot(h, w2_ref[...], preferred_element_type=jnp.float32) \
        + bb2_ref[...]
    mu = y.mean(-1, keepdims=True)
    var = ((y - mu) ** 2).mean(-1, keepdims=True)
    o_ref[0] = (y - mu) / jnp.sqrt(var + 1e-5) * g2_ref[...] + b2_ref[...]


def _tail(sel, bag, p):
    grid = (B * LQP // RB2,)
    sel = sel.reshape(B * LQP // RB2, RB2, D)
    bag = bag.reshape(B * LQP // RB2, RB2, D)
    vec = lambda: pl.BlockSpec((D,), lambda i: (0,))
    mat = lambda s: pl.BlockSpec(s, lambda i: (0, 0))
    out = pl.pallas_call(
        _tail_body,
        out_shape=jax.ShapeDtypeStruct((B * LQP // RB2, RB2, D), jnp.float32),
        grid=grid,
        in_specs=[
            pl.BlockSpec((1, RB2, D), lambda i: (i, 0, 0)),
            pl.BlockSpec((1, RB2, D), lambda i: (i, 0, 0)),
            mat((D, D)), vec(), vec(), vec(),
            mat((D, DFF)), pl.BlockSpec((DFF,), lambda i: (0,)),
            mat((DFF, D)), vec(), vec(), vec(),
        ],
        out_specs=pl.BlockSpec((1, RB2, D), lambda i: (i, 0, 0)),
    )(sel, bag, p['out_proj_w'].T, p['out_proj_b'],
      p['norm1_g'], p['norm1_b'],
      p['lin1_w'].T, p['lin1_b'], p['lin2_w'].T, p['lin2_b'],
      p['norm2_g'], p['norm2_b'])
    return out.reshape(B, LQP, D)


def kernel(src, pos, src_shape, src_start_idx, ref_windows, score_mask,
           params):
    p = params
    Bq = src.shape[0]
    sel_score, indices = jax.lax.top_k(score_mask, FG)
    select_ref = _gat(ref_windows, indices)
    fidx = indices + (jnp.arange(B, dtype=indices.dtype) * N)[:, None]
    fidx = jnp.pad(fidx, ((0, 0), (0, LQP - FG)))
    sel_src_p, sel_pos_p = _sc_select(
        src.reshape(B * N, D), pos.reshape(B * N, D), fidx.reshape(B * LQP))
    select_src = sel_src_p.reshape(B, LQP, D)
    select_pos = sel_pos_p.reshape(B, LQP, D)
    query_src = select_src[:, :QN]
    query_pos = select_pos[:, :QN]
    q = query_src + query_pos
    q2 = _mha_small(q, q, query_src, p)
    query_src = _ln(query_src + q2, p['query_norm_g'], p['query_norm_b'])
    select_src = jnp.concatenate([query_src, select_src[:, QN:]], axis=1)

    v = (src.reshape(B * N, D) @ p['value_proj_w'].T + p['value_proj_b'])
    vg = jnp.pad(v.reshape(B, HGRID, WGRID, NH, DH),
                 ((0, 0), (1, 2), (1, 2), (0, 0), (0, 0)))
    vn = jnp.stack([vg[:, :-1, :-1], vg[:, :-1, 1:],
                    vg[:, 1:, :-1], vg[:, 1:, 1:]], axis=4)
    vt = vn.reshape(B * PG * PG * NH, 4 * DH)

    query = select_src + select_pos
    refp = jnp.pad(select_ref, ((0, 0), (0, LQP - FG), (0, 128 - 7)))
    idxs, wgts = _idx_wgt(query, refp, p)
    bag = _sc_bag(vt, idxs.reshape(TOTQ, NP_), wgts.reshape(TOTQ * NPTS))
    bag = bag.reshape(B, LQP, D)

    y = _tail(select_src, bag, p)[:, :FG]

    bidx = jnp.arange(Bq)[:, None]
    return src.at[bidx, indices].set(y)
